# baseline (speedup 1.0000x reference)
import functools

import jax
import jax.numpy as jnp
from jax import lax
from jax.experimental import pallas as pl
from jax.experimental.pallas import tpu as pltpu
from jax.experimental.pallas import tpu_sc as plsc

_NC = 2
_NS = 16
_L = 16
_R = 49920
_CK = 2048
_DH = 4
_U = 4


def kernel(text_indices, context_indices, w_in, w_out):
    B = text_indices.shape[0]
    N, E = w_in.shape
    assert E == _NC * _NS and B % _CK == 0
    tail = N - 2 * _R
    assert 0 < tail <= 256
    b_per_t = B // _NS
    mesh = plsc.VectorSubcoreMesh(core_axis_name="c", subcore_axis_name="s")

    @functools.partial(
        pl.kernel,
        mesh=mesh,
        out_type=jax.ShapeDtypeStruct((_NC, B), jnp.float32),
        compiler_params=pltpu.CompilerParams(needs_layout_passes=False),
        scratch_types=[
            pltpu.VMEM((_R + tail,), jnp.float32),
            pltpu.VMEM((_R + tail,), jnp.float32),
            pltpu.VMEM((tail,), jnp.float32),
            pltpu.VMEM((_CK,), jnp.int32),
            pltpu.VMEM((B,), jnp.float32),
            pltpu.VMEM((_CK,), jnp.float32),
            pltpu.VMEM((_DH, B // _NS), jnp.float32),
            pltpu.HBM((_NC, _NS, B), jnp.float32),
            pltpu.SemaphoreType.DMA,
            pltpu.SemaphoreType.DMA,
        ],
    )
    def sc_kernel(ti_hbm, ci_hbm, wt_in_hbm, wt_out_hbm, out_hbm,
                  row_a, row_b, tail_v, idx_v, xfull, pc_v, pbuf, p_hbm,
                  sem_a, sem_b):
        c = lax.axis_index("c")
        t = lax.axis_index("s")
        d = c * _NS + t

        def stage(tbl_hbm):
            return (pltpu.async_copy(tbl_hbm.at[d, pl.ds(0, _R)],
                                     row_a.at[pl.ds(0, _R)], sem_a),
                    pltpu.async_copy(tbl_hbm.at[d, pl.ds(_R, _R)],
                                     row_b.at[pl.ds(0, _R)], sem_b),
                    pltpu.async_copy(tbl_hbm.at[d, pl.ds(2 * _R, tail)],
                                     tail_v, sem_b))

        def wait_and_merge(cps):
            for cp in cps:
                cp.wait()
            for k in range(tail // _L):
                row_b[pl.ds(_R + k * _L, _L)] = tail_v[pl.ds(k * _L, _L)]

        def gather2(iv):
            m0 = iv < _R
            g0 = plsc.load_gather(row_a, [iv], mask=m0)
            m1 = iv >= _R
            g1 = plsc.load_gather(row_b, [iv - _R], mask=m1)
            return jnp.where(m0, g0, 0.0) + jnp.where(m1, g1, 0.0)

        cps = stage(wt_in_hbm)
        wait_and_merge(cps)
        for k in range(B // _CK):
            kbase = k * _CK
            pltpu.sync_copy(ti_hbm.at[pl.ds(kbase, _CK)], idx_v)

            @plsc.parallel_loop(0, _CK, step=_L, unroll=_U)
            def xbody(i):
                xfull[pl.ds(kbase + i, _L)] = gather2(idx_v[pl.ds(i, _L)])

        cps = stage(wt_out_hbm)
        wait_and_merge(cps)
        for k in range(B // _CK):
            kbase = k * _CK
            pltpu.sync_copy(ci_hbm.at[pl.ds(kbase, _CK)], idx_v)

            @plsc.parallel_loop(0, _CK, step=_L, unroll=_U)
            def cbody(i):
                cv = gather2(idx_v[pl.ds(i, _L)])
                pc_v[pl.ds(i, _L)] = cv * xfull[pl.ds(kbase + i, _L)]

            pltpu.sync_copy(pc_v, p_hbm.at[c, t, pl.ds(kbase, _CK)])

        plsc.subcore_barrier()

        bbase = t * b_per_t
        for dchunk in range(_NS // _DH):
            dbase = dchunk * _DH
            pltpu.sync_copy(
                p_hbm.at[c, pl.ds(dbase, _DH), pl.ds(bbase, b_per_t)], pbuf)

            @plsc.parallel_loop(0, b_per_t, step=_L, unroll=_U)
            def rbody(v):
                sl = pl.ds(v, _L)
                acc = pbuf[0, sl]
                for dd in range(1, _DH):
                    acc = acc + pbuf[dd, sl]
                if dchunk:
                    acc = acc + xfull[sl]
                xfull[sl] = acc

        pltpu.sync_copy(xfull.at[pl.ds(0, b_per_t)],
                        out_hbm.at[c, pl.ds(bbase, b_per_t)])

    partials = sc_kernel(text_indices, context_indices, w_in.T, w_out.T)
    return partials[0] + partials[1]


# speedup vs baseline: 1.0923x; 1.0923x over previous
import functools

import jax
import jax.numpy as jnp
from jax import lax
from jax.experimental import pallas as pl
from jax.experimental.pallas import tpu as pltpu
from jax.experimental.pallas import tpu_sc as plsc

_NC = 2
_NS = 16
_L = 16
_R = 49920
_CK = 2048
_DH = 4
_U = 4


def kernel(text_indices, context_indices, w_in, w_out):
    B = text_indices.shape[0]
    N, E = w_in.shape
    assert E == _NC * _NS and B % _CK == 0
    tail = N - 2 * _R
    assert 0 < tail <= 256
    nck = B // _CK
    b_per_t = B // _NS
    mesh = plsc.VectorSubcoreMesh(core_axis_name="c", subcore_axis_name="s")

    @functools.partial(
        pl.kernel,
        mesh=mesh,
        out_type=jax.ShapeDtypeStruct((_NC, B), jnp.float32),
        compiler_params=pltpu.CompilerParams(needs_layout_passes=False),
        scratch_types=[
            pltpu.VMEM((_R + tail,), jnp.float32),
            pltpu.VMEM((_R + tail,), jnp.float32),
            pltpu.VMEM((tail,), jnp.float32),
            pltpu.VMEM((_CK,), jnp.int32),
            pltpu.VMEM((_CK,), jnp.int32),
            pltpu.VMEM((B,), jnp.float32),
            pltpu.VMEM((_CK,), jnp.float32),
            pltpu.VMEM((_CK,), jnp.float32),
            pltpu.VMEM((_DH, B // _NS), jnp.float32),
            pltpu.HBM((_NC, _NS, B), jnp.float32),
            pltpu.SemaphoreType.DMA,
            pltpu.SemaphoreType.DMA,
            pltpu.SemaphoreType.DMA,
            pltpu.SemaphoreType.DMA,
        ],
    )
    def sc_kernel(ti_hbm, ci_hbm, wt_in_hbm, wt_out_hbm, out_hbm,
                  row_a, row_b, tail_v, idx_v0, idx_v1, xfull, pc_v0, pc_v1,
                  pbuf, p_hbm, sem_a, sem_b, sem_i, sem_p):
        c = lax.axis_index("c")
        t = lax.axis_index("s")
        d = c * _NS + t

        def stage(tbl_hbm):
            return (pltpu.async_copy(tbl_hbm.at[d, pl.ds(0, _R)],
                                     row_a.at[pl.ds(0, _R)], sem_a),
                    pltpu.async_copy(tbl_hbm.at[d, pl.ds(_R, _R)],
                                     row_b.at[pl.ds(0, _R)], sem_b),
                    pltpu.async_copy(tbl_hbm.at[d, pl.ds(2 * _R, tail)],
                                     tail_v, sem_b))

        def wait_and_merge(cps):
            for cp in cps:
                cp.wait()
            for k in range(tail // _L):
                row_b[pl.ds(_R + k * _L, _L)] = tail_v[pl.ds(k * _L, _L)]

        def gather2(iv):
            m0 = iv < _R
            g0 = plsc.load_gather(row_a, [iv], mask=m0)
            m1 = iv >= _R
            g1 = plsc.load_gather(row_b, [iv - _R], mask=m1)
            return jnp.where(m0, g0, 0.0) + jnp.where(m1, g1, 0.0)

        idxbufs = (idx_v0, idx_v1)
        pcbufs = (pc_v0, pc_v1)

        def idx_fetch(i_hbm, k):
            return pltpu.async_copy(i_hbm.at[pl.ds(k * _CK, _CK)],
                                    idxbufs[k % 2], sem_i)

        cps = stage(wt_in_hbm)
        icp = idx_fetch(ti_hbm, 0)
        wait_and_merge(cps)
        for k in range(nck):
            icp.wait()
            icp = idx_fetch(ti_hbm, k + 1) if k + 1 < nck else None
            kbase = k * _CK
            ibuf = idxbufs[k % 2]

            @plsc.parallel_loop(0, _CK, step=_L, unroll=_U)
            def xbody(i):
                xfull[pl.ds(kbase + i, _L)] = gather2(ibuf[pl.ds(i, _L)])

        cps = stage(wt_out_hbm)
        icp = idx_fetch(ci_hbm, 0)
        wait_and_merge(cps)
        pushes = [None, None]
        for k in range(nck):
            icp.wait()
            icp = idx_fetch(ci_hbm, k + 1) if k + 1 < nck else None
            kbase = k * _CK
            ibuf = idxbufs[k % 2]
            pbuf2 = pcbufs[k % 2]
            if pushes[k % 2] is not None:
                pushes[k % 2].wait()

            @plsc.parallel_loop(0, _CK, step=_L, unroll=_U)
            def cbody(i):
                cv = gather2(ibuf[pl.ds(i, _L)])
                pbuf2[pl.ds(i, _L)] = cv * xfull[pl.ds(kbase + i, _L)]

            pushes[k % 2] = pltpu.async_copy(
                pbuf2, p_hbm.at[c, t, pl.ds(kbase, _CK)], sem_p)
        for push in pushes:
            if push is not None:
                push.wait()

        plsc.subcore_barrier()

        bbase = t * b_per_t
        for dchunk in range(_NS // _DH):
            dbase = dchunk * _DH
            pltpu.sync_copy(
                p_hbm.at[c, pl.ds(dbase, _DH), pl.ds(bbase, b_per_t)], pbuf)

            @plsc.parallel_loop(0, b_per_t, step=_L, unroll=_U)
            def rbody(v):
                sl = pl.ds(v, _L)
                acc = pbuf[0, sl]
                for dd in range(1, _DH):
                    acc = acc + pbuf[dd, sl]
                if dchunk:
                    acc = acc + xfull[sl]
                xfull[sl] = acc

        pltpu.sync_copy(xfull.at[pl.ds(0, b_per_t)],
                        out_hbm.at[c, pl.ds(bbase, b_per_t)])

    partials = sc_kernel(text_indices, context_indices, w_in.T, w_out.T)
    return partials[0] + partials[1]
